# baseline (speedup 1.0000x reference)
import jax
import jax.numpy as jnp
from jax import lax
from jax.experimental import pallas as pl
from jax.experimental.pallas import tpu as pltpu
from jax.experimental.pallas import tpu_sc as plsc

D = 128
LANES = 16
NUM_WORKERS = 32
CHUNK = 64
SETS = 5
LOOKAHEAD = 3
ROW_UNROLL = 8


def _sc_body(xs_hbm, e0_hbm, e1_hbm, e2_hbm, out_hbm, *scratch):
    idxs = scratch[0:SETS]
    bufs = [scratch[SETS + 3 * s: SETS + 3 * s + 3] for s in range(SETS)]
    isems = scratch[SETS + 3 * SETS: 2 * SETS + 3 * SETS]
    gsems = scratch[2 * SETS + 3 * SETS: 3 * SETS + 3 * SETS]
    osems = scratch[3 * SETS + 3 * SETS: 4 * SETS + 3 * SETS]

    n_tokens = out_hbm.shape[0]
    per_worker = n_tokens // NUM_WORKERS
    n_chunks = per_worker // CHUNK
    wid = lax.axis_index("s") * 2 + lax.axis_index("c")
    base = wid * per_worker
    idx_base = wid * n_chunks

    def idx_desc(g, s):
        return pltpu.make_async_copy(xs_hbm.at[idx_base + g], idxs[s], isems[s])

    def out_desc(g, s):
        return pltpu.make_async_copy(
            bufs[s][0], out_hbm.at[pl.ds(base + g * CHUNK, CHUNK)], osems[s])

    def gather_descs(s):
        return (pltpu.make_async_copy(e0_hbm.at[idxs[s].at[0]], bufs[s][0], gsems[s]),
                pltpu.make_async_copy(e1_hbm.at[idxs[s].at[1]], bufs[s][1], gsems[s]),
                pltpu.make_async_copy(e2_hbm.at[idxs[s].at[2]], bufs[s][2], gsems[s]))

    def fire_gathers(s):
        for c in gather_descs(s):
            c.start()

    def wait_gathers(s):
        for c in gather_descs(s):
            c.wait()

    for s in range(SETS):
        idx_desc(s, s).start()
    for s in range(LOOKAHEAD):
        idx_desc(s, s).wait()
        fire_gathers(s)

    def quad_body(gg, carry):
        for b in range(SETS):
            g = gg * SETS + b
            b0, b1, b2 = bufs[b]
            wait_gathers(b)

            @pl.when(g + SETS < n_chunks)
            def _():
                idx_desc(g + SETS, b).start()

            nxt = (b + LOOKAHEAD) % SETS
            lag = SETS - LOOKAHEAD

            @pl.when(jnp.logical_and(g + LOOKAHEAD < n_chunks, g >= lag))
            def _():
                out_desc(g - lag, nxt).wait()

            @pl.when(g + LOOKAHEAD < n_chunks)
            def _():
                idx_desc(g + LOOKAHEAD, nxt).wait()
                fire_gathers(nxt)

            def add_rows(i, c):
                for r in range(ROW_UNROLL):
                    row = i * ROW_UNROLL + r
                    for j in range(D // LANES):
                        sl = pl.ds(j * LANES, LANES)
                        plsc.addupdate(b0.at[row, sl], b1[row, sl] + b2[row, sl])
                return c

            lax.fori_loop(0, CHUNK // ROW_UNROLL, add_rows, 0)
            out_desc(g, b).start()
        return carry

    lax.fori_loop(0, n_chunks // SETS, quad_body, 0)

    for k in range(SETS):
        out_desc(n_chunks - SETS + k, (n_chunks - SETS + k) % SETS).wait()


def _make_kernel(n_tokens):
    mesh = plsc.VectorSubcoreMesh(core_axis_name="c", subcore_axis_name="s")
    scratch = (
        [pltpu.VMEM((3, CHUNK), jnp.int32) for _ in range(SETS)]
        + [pltpu.VMEM((CHUNK, D), jnp.float32) for _ in range(3 * SETS)]
        + [pltpu.SemaphoreType.DMA for _ in range(3 * SETS)]
    )
    return pl.kernel(
        _sc_body,
        out_type=jax.ShapeDtypeStruct((n_tokens, D), jnp.float32),
        mesh=mesh,
        scratch_types=scratch,
    )


@jax.jit
def kernel(x, e0, e1, e2):
    b, t, _ = x.shape
    n_tokens = b * t
    per_worker = n_tokens // NUM_WORKERS
    n_chunks = per_worker // CHUNK
    xs = (x.astype(jnp.int32)
          .reshape(NUM_WORKERS, n_chunks, CHUNK, 3)
          .transpose(0, 1, 3, 2)
          .reshape(NUM_WORKERS * n_chunks, 3, CHUNK))
    out = _make_kernel(n_tokens)(xs, e0, e1, e2)
    return out.reshape(b, t, D)


# speedup vs baseline: 1.0042x; 1.0042x over previous
import jax
import jax.numpy as jnp
from jax import lax
from jax.experimental import pallas as pl
from jax.experimental.pallas import tpu as pltpu
from jax.experimental.pallas import tpu_sc as plsc

D = 128
LANES = 16
NUM_WORKERS = 32
CHUNK = 64
SETS = 4
ROW_UNROLL = 8


def _sc_body(xs_hbm, e0_hbm, e1_hbm, e2_hbm, out_hbm, *scratch):
    idxs = scratch[0:SETS]
    bufs = [scratch[SETS + 3 * s: SETS + 3 * s + 3] for s in range(SETS)]
    isems = scratch[SETS + 3 * SETS: 2 * SETS + 3 * SETS]
    gsems = scratch[2 * SETS + 3 * SETS: 3 * SETS + 3 * SETS]
    osems = scratch[3 * SETS + 3 * SETS: 4 * SETS + 3 * SETS]

    n_tokens = out_hbm.shape[0]
    per_worker = n_tokens // NUM_WORKERS
    n_chunks = per_worker // CHUNK
    wid = lax.axis_index("s") * 2 + lax.axis_index("c")
    base = wid * per_worker
    idx_base = wid * n_chunks

    def idx_desc(g, s):
        return pltpu.make_async_copy(xs_hbm.at[idx_base + g], idxs[s], isems[s])

    def out_desc(g, s):
        return pltpu.make_async_copy(
            bufs[s][0], out_hbm.at[pl.ds(base + g * CHUNK, CHUNK)], osems[s])

    def gather_descs(s):
        return (pltpu.make_async_copy(e0_hbm.at[idxs[s].at[0]], bufs[s][0], gsems[s]),
                pltpu.make_async_copy(e1_hbm.at[idxs[s].at[1]], bufs[s][1], gsems[s]),
                pltpu.make_async_copy(e2_hbm.at[idxs[s].at[2]], bufs[s][2], gsems[s]))

    def fire_gathers(s):
        for c in gather_descs(s):
            c.start()

    def wait_gathers(s):
        for c in gather_descs(s):
            c.wait()

    for s in range(SETS):
        idx_desc(s, s).start()
    idx_desc(0, 0).wait()
    fire_gathers(0)
    idx_desc(1, 1).wait()
    fire_gathers(1)

    def quad_body(gg, carry):
        for b in range(SETS):
            g = gg * SETS + b
            b0, b1, b2 = bufs[b]
            wait_gathers(b)

            @pl.when(g + SETS < n_chunks)
            def _():
                idx_desc(g + SETS, b).start()

            nxt = (b + 2) % SETS

            @pl.when(jnp.logical_and(g + 2 < n_chunks, g >= 2))
            def _():
                out_desc(g - 2, nxt).wait()

            @pl.when(g + 2 < n_chunks)
            def _():
                idx_desc(g + 2, nxt).wait()
                fire_gathers(nxt)

            def add_rows(i, c):
                for r in range(ROW_UNROLL):
                    row = i * ROW_UNROLL + r
                    for j in range(D // LANES):
                        sl = pl.ds(j * LANES, LANES)
                        plsc.addupdate(b0.at[row, sl], b1[row, sl] + b2[row, sl])
                return c

            lax.fori_loop(0, CHUNK // ROW_UNROLL, add_rows, 0)
            out_desc(g, b).start()
        return carry

    lax.fori_loop(0, n_chunks // SETS, quad_body, 0)

    for k in range(SETS):
        out_desc(n_chunks - SETS + k, (n_chunks - SETS + k) % SETS).wait()


def _make_kernel(n_tokens):
    mesh = plsc.VectorSubcoreMesh(core_axis_name="c", subcore_axis_name="s")
    scratch = (
        [pltpu.VMEM((3, CHUNK), jnp.int32) for _ in range(SETS)]
        + [pltpu.VMEM((CHUNK, D), jnp.float32) for _ in range(3 * SETS)]
        + [pltpu.SemaphoreType.DMA for _ in range(3 * SETS)]
    )
    return pl.kernel(
        _sc_body,
        out_type=jax.ShapeDtypeStruct((n_tokens, D), jnp.float32),
        mesh=mesh,
        scratch_types=scratch,
    )


@jax.jit
def kernel(x, e0, e1, e2):
    b, t, _ = x.shape
    n_tokens = b * t
    per_worker = n_tokens // NUM_WORKERS
    n_chunks = per_worker // CHUNK
    xs = (x.astype(jnp.int32)
          .reshape(NUM_WORKERS, n_chunks, CHUNK, 3)
          .transpose(0, 1, 3, 2)
          .reshape(NUM_WORKERS * n_chunks, 3, CHUNK))
    out = _make_kernel(n_tokens)(xs, e0, e1, e2)
    return out.reshape(b, t, D)
